# baseline (speedup 1.0000x reference)
import functools

import jax
import jax.numpy as jnp
import numpy as np
from jax import lax
from jax.experimental import pallas as pl
from jax.experimental.pallas import tpu as pltpu
from jax.experimental.pallas import tpu_sc as plsc

B, C, K = 64, 512, 1024
H, W = 32, 32
HW = H * W
FG = 128
BG = 128
INT_MIN = np.int32(-(2 ** 31))
POS_MASK = np.int32(2 ** 31 - 1)

NUM_SC_CORES = 2
NUM_SUBCORES = 16
NUM_WORKERS = NUM_SC_CORES * NUM_SUBCORES
B_PER_W = B // NUM_WORKERS


def _bit(b):
    return INT_MIN if b == 31 else np.int32(1 << b)


def _low_mask(b):
    return np.int32((1 << b) - 1)


A1_ROWS = 8


def _sim_topk_body(att_ref, aud_ref, sim_ref, thr_ref, ngt_ref, pos_ref,
                   neg_ref, simacc_ref):
    step = pl.program_id(0)
    for i in range(A1_ROWS):
        f = att_ref[i]
        a = aud_ref[i]
        dot = jnp.sum(f * a, axis=0, keepdims=True)
        ss = jnp.sum(f * f, axis=0, keepdims=True)
        row = dot / jnp.maximum(jnp.sqrt(ss), 1e-12)
        sim_ref[pl.ds(i, 1), :] = row
        simacc_ref[pl.ds(step * A1_ROWS + i, 1), :] = row

    @pl.when(step == B // A1_ROWS - 1)
    def _():
        sim = simacc_ref[...]
        ibits = lax.bitcast_convert_type(sim, jnp.int32)
        key = jnp.where(ibits >= 0, ibits, ibits ^ POS_MASK)

        prefix = jnp.zeros((B, 1), jnp.int32)
        for b in range(31, -1, -1):
            trial = prefix | _bit(b)
            cnt = jnp.sum((key >= (trial ^ INT_MIN)).astype(jnp.float32),
                          axis=1, keepdims=True)
            prefix = jnp.where(cnt >= float(FG), trial, prefix)
        s_top = prefix ^ INT_MIN

        prefixb = jnp.zeros((B, 1), jnp.int32)
        for b in range(31, -1, -1):
            trial = prefixb | _low_mask(b)
            cnt = jnp.sum((key <= (trial ^ INT_MIN)).astype(jnp.float32),
                          axis=1, keepdims=True)
            prefixb = jnp.where(cnt >= float(BG), prefixb, prefixb | _bit(b))
        s_bot = prefixb ^ INT_MIN

        gt = key > s_top
        n_gt = jnp.sum(gt.astype(jnp.float32), axis=1, keepdims=True)
        sum_gt = jnp.sum(jnp.where(gt, sim, 0.0), axis=1, keepdims=True)
        v_top = lax.bitcast_convert_type(
            jnp.where(s_top >= 0, s_top, s_top ^ POS_MASK), jnp.float32)
        pos = (sum_gt + (FG - n_gt) * v_top) / FG

        lt = key < s_bot
        n_lt = jnp.sum(lt.astype(jnp.float32), axis=1, keepdims=True)
        sum_lt = jnp.sum(jnp.where(lt, sim, 0.0), axis=1, keepdims=True)
        v_bot = lax.bitcast_convert_type(
            jnp.where(s_bot >= 0, s_bot, s_bot ^ POS_MASK), jnp.float32)
        neg = (sum_lt + (BG - n_lt) * v_bot) / BG

        thr_ref[...] = jnp.broadcast_to(v_top, (B, 16))
        ngt_ref[...] = jnp.broadcast_to(n_gt.astype(jnp.int32), (B, 16))
        pos_ref[...] = jnp.broadcast_to(pos, (B, 16))
        neg_ref[...] = jnp.broadcast_to(neg, (B, 16))


def _run_stage_a(att_feat, aud_feat):
    aud3 = aud_feat.reshape(B, C, 1)
    return pl.pallas_call(
        _sim_topk_body,
        grid=(B // A1_ROWS,),
        in_specs=[
            pl.BlockSpec((A1_ROWS, C, K), lambda b: (b, 0, 0)),
            pl.BlockSpec((A1_ROWS, C, 1), lambda b: (b, 0, 0)),
        ],
        out_specs=[
            pl.BlockSpec((A1_ROWS, K), lambda b: (b, 0)),
            pl.BlockSpec((B, 16), lambda b: (0, 0)),
            pl.BlockSpec((B, 16), lambda b: (0, 0)),
            pl.BlockSpec((B, 16), lambda b: (0, 0)),
            pl.BlockSpec((B, 16), lambda b: (0, 0)),
        ],
        out_shape=[
            jax.ShapeDtypeStruct((B, K), jnp.float32),
            jax.ShapeDtypeStruct((B, 16), jnp.float32),
            jax.ShapeDtypeStruct((B, 16), jnp.int32),
            jax.ShapeDtypeStruct((B, 16), jnp.float32),
            jax.ShapeDtypeStruct((B, 16), jnp.float32),
        ],
        scratch_shapes=[pltpu.VMEM((B, K), jnp.float32)],
    )(att_feat, aud3)


def _sc_mask_body(sim_hbm, thr_hbm, ngt_hbm, out_hbm,
                  sim_v, thr_v, ngt_v, mask_v, sem):
    del sem
    cid = lax.axis_index("c")
    sid = lax.axis_index("s")
    wid = cid * NUM_SUBCORES + sid
    zeros16i = jnp.zeros((16,), jnp.int32)
    ones16f = jnp.full((16,), 1.0, jnp.float32)
    zeros16f = jnp.zeros((16,), jnp.float32)

    for i in range(B_PER_W):
        b = wid * B_PER_W + i
        pltpu.sync_copy(sim_hbm.at[b], sim_v)
        pltpu.sync_copy(thr_hbm.at[b], thr_v)
        pltpu.sync_copy(ngt_hbm.at[b], ngt_v)
        thr = thr_v[...]
        ngt = ngt_v[...]

        ceq = zeros16i
        for j in range(K // 16):
            svec = sim_v[pl.ds(j * 16, 16)]
            m_gt = svec > thr
            m_eq = svec == thr
            incl_e = plsc.cumsum(m_eq.astype(jnp.int32))
            tie_rank = ngt + ceq + incl_e - 1
            m_sel = jnp.logical_or(
                m_gt, jnp.logical_and(m_eq, tie_rank < FG))
            mask_v[pl.ds(j * 16, 16)] = jnp.where(m_sel, ones16f, zeros16f)
            ceq = ceq + plsc.all_reduce_population_count(m_eq)

        pltpu.sync_copy(mask_v, out_hbm.at[b])


def _run_stage_b1(sim, thr, ngt):
    mesh = plsc.VectorSubcoreMesh(core_axis_name="c", subcore_axis_name="s")
    fn = functools.partial(
        pl.kernel,
        out_type=jax.ShapeDtypeStruct((B, K), jnp.float32),
        mesh=mesh,
        compiler_params=pltpu.CompilerParams(needs_layout_passes=False),
        scratch_types=[
            pltpu.VMEM((K,), jnp.float32),
            pltpu.VMEM((16,), jnp.float32),
            pltpu.VMEM((16,), jnp.int32),
            pltpu.VMEM((K,), jnp.float32),
            pltpu.SemaphoreType.DMA,
        ],
    )(_sc_mask_body)
    return fn(sim, thr, ngt)


B_SC = 16
B_TC = B - B_SC
ROWS_PER_W = B_SC * HW // NUM_WORKERS
SC_CHUNK = 32


def _sc_reduce_body(mask_hbm, hm_hbm, out_hbm,
                    mask_v, buf0_v, buf1_v, comb_v, sem0, sem1):
    cid = lax.axis_index("c")
    sid = lax.axis_index("s")
    wid = cid * NUM_SUBCORES + sid
    bs = B_TC + wid // 2
    half = wid % 2
    base = bs * HW + half * ROWS_PER_W
    zeros16f = jnp.zeros((16,), jnp.float32)
    lane0 = lax.iota(jnp.int32, 16) == 0

    pltpu.sync_copy(mask_hbm.at[bs], mask_v)

    bufs = (buf0_v, buf1_v)
    sems = (sem0, sem1)
    nch = ROWS_PER_W // SC_CHUNK
    descs = [None] * nch
    descs[0] = pltpu.async_copy(hm_hbm.at[pl.ds(base, SC_CHUNK)], buf0_v,
                                sem0)
    for ch in range(nch):
        cur = bufs[ch % 2]
        descs[ch].wait()
        if ch + 1 < nch:
            descs[ch + 1] = pltpu.async_copy(
                hm_hbm.at[pl.ds(base + (ch + 1) * SC_CHUNK, SC_CHUNK)],
                bufs[(ch + 1) % 2], sems[(ch + 1) % 2])

        for rb in (0, 16):
            def jbody(j, accs, cur=cur, rb=rb):
                mj = mask_v[pl.ds(j * 16, 16)]
                return tuple(
                    accs[r] + cur[rb + r, pl.ds(j * 16, 16)] * mj
                    for r in range(16))

            accs = lax.fori_loop(0, K // 16, jbody, (zeros16f,) * 16)
            for r in range(16):
                pix = zeros16f + jnp.sum(accs[r])
                idx = jnp.full((16,), ch * SC_CHUNK + rb + r, jnp.int32)
                plsc.store_scatter(comb_v, [idx], pix, mask=lane0)

    pltpu.sync_copy(comb_v,
                    out_hbm.at[bs - B_TC, pl.ds(half * ROWS_PER_W,
                                                ROWS_PER_W)])


def _run_stage_b3(mask, hm_flat):
    mesh = plsc.VectorSubcoreMesh(core_axis_name="c", subcore_axis_name="s")
    fn = functools.partial(
        pl.kernel,
        out_type=jax.ShapeDtypeStruct((B_SC, HW), jnp.float32),
        mesh=mesh,
        compiler_params=pltpu.CompilerParams(needs_layout_passes=False),
        scratch_types=[
            pltpu.VMEM((K,), jnp.float32),
            pltpu.VMEM((SC_CHUNK, K), jnp.float32),
            pltpu.VMEM((SC_CHUNK, K), jnp.float32),
            pltpu.VMEM((ROWS_PER_W,), jnp.float32),
            pltpu.SemaphoreType.DMA,
            pltpu.SemaphoreType.DMA,
        ],
    )(_sc_reduce_body)
    return fn(mask, hm_flat)


B2_ROWS = 4


def _hm_reduce_body(hm_ref, mask_ref, out_ref):
    for i in range(B2_ROWS):
        hm = hm_ref[i].reshape(HW, K)
        m = mask_ref[i]
        comb = lax.dot_general(m, hm, (((1,), (1,)), ((), ())),
                               preferred_element_type=jnp.float32)
        out_ref[pl.ds(i, 1), :, :] = comb.reshape(1, 1, HW)


def _run_stage_b2(hm_t, mask):
    return pl.pallas_call(
        _hm_reduce_body,
        grid=(B_TC // B2_ROWS,),
        in_specs=[
            pl.BlockSpec((B2_ROWS, H, W, K), lambda b: (b, 0, 0, 0)),
            pl.BlockSpec((B2_ROWS, 1, K), lambda b: (b, 0, 0)),
        ],
        out_specs=pl.BlockSpec((B2_ROWS, 1, HW), lambda b: (b, 0, 0)),
        out_shape=jax.ShapeDtypeStruct((B_TC, 1, HW), jnp.float32),
    )(hm_t, mask.reshape(B, 1, K))


def _loss_body(pos_ref, neg_ref, comb_ref, scc_ref, av_ref, dis_ref, div_ref):
    p = pos_ref[:, 0:1]
    n = neg_ref[:, 0:1]
    mx = jnp.maximum(p, n)
    lse = mx + jnp.log(jnp.exp(p - mx) + jnp.exp(n - mx))
    dis_ref[...] = jnp.mean(lse - p).reshape(1, 1)

    c = jnp.concatenate([comb_ref[:, 0, :], scc_ref[...]],
                        axis=0) * (1.0 / FG)
    a = av_ref[...]
    cm = jnp.max(c, axis=1, keepdims=True)
    ce = jnp.exp(c - cm)
    cz = jnp.sum(ce, axis=1, keepdims=True)
    att = ce / cz
    log_att = (c - cm) - jnp.log(cz)
    am = jnp.max(a, axis=1, keepdims=True)
    ae = jnp.exp(a - am)
    az = jnp.sum(ae, axis=1, keepdims=True)
    avd = ae / az
    log_av = (a - am) - jnp.log(az)
    logm = jnp.log(0.5 * (att + avd))
    div = (jnp.sum(att * (log_att - logm)) +
           jnp.sum(avd * (log_av - logm))) / (2.0 * B)
    div_ref[...] = div.reshape(1, 1)


def _run_stage_c(pos, neg, comb, scc, av):
    return pl.pallas_call(
        _loss_body,
        out_shape=[
            jax.ShapeDtypeStruct((1, 1), jnp.float32),
            jax.ShapeDtypeStruct((1, 1), jnp.float32),
        ],
    )(pos, neg, comb, scc, av)


def kernel(att_feat, aud_feat, att_heatmaps, av_heatmaps):
    sim, thr, ngt, pos, neg = _run_stage_a(att_feat, aud_feat)
    mask = _run_stage_b1(sim, thr, ngt)
    hm_t = jnp.transpose(att_heatmaps, (0, 2, 3, 1))
    hm_flat = hm_t.reshape(B * HW, K)
    scc = _run_stage_b3(mask, hm_flat)
    comb = _run_stage_b2(hm_t, mask)
    dis, div = _run_stage_c(pos, neg, comb, scc, av_heatmaps.reshape(B, HW))
    return dis.reshape(()), div.reshape(())


# speedup vs baseline: 1.0174x; 1.0174x over previous
import functools

import jax
import jax.numpy as jnp
import numpy as np
from jax import lax
from jax.experimental import pallas as pl
from jax.experimental.pallas import tpu as pltpu
from jax.experimental.pallas import tpu_sc as plsc

B, C, K = 64, 512, 1024
H, W = 32, 32
HW = H * W
FG = 128
BG = 128
INT_MIN = np.int32(-(2 ** 31))
POS_MASK = np.int32(2 ** 31 - 1)

NUM_SC_CORES = 2
NUM_SUBCORES = 16
NUM_WORKERS = NUM_SC_CORES * NUM_SUBCORES
B_PER_W = B // NUM_WORKERS


def _bit(b):
    return INT_MIN if b == 31 else np.int32(1 << b)


def _low_mask(b):
    return np.int32((1 << b) - 1)


A1_ROWS = 8


def _sim_topk_body(att_ref, aud_ref, sim_ref, thr_ref, ngt_ref, pos_ref,
                   neg_ref, simacc_ref):
    step = pl.program_id(0)
    for i in range(A1_ROWS):
        f = att_ref[i]
        a = aud_ref[i]
        dot = jnp.sum(f * a, axis=0, keepdims=True)
        ss = jnp.sum(f * f, axis=0, keepdims=True)
        row = dot / jnp.maximum(jnp.sqrt(ss), 1e-12)
        sim_ref[pl.ds(i, 1), :] = row
        simacc_ref[pl.ds(step * A1_ROWS + i, 1), :] = row

    @pl.when(step == B // A1_ROWS - 1)
    def _():
        sim = simacc_ref[...]
        ibits = lax.bitcast_convert_type(sim, jnp.int32)
        key = jnp.where(ibits >= 0, ibits, ibits ^ POS_MASK)

        prefix = jnp.zeros((B, 1), jnp.int32)
        for b in range(31, -1, -1):
            trial = prefix | _bit(b)
            cnt = jnp.sum((key >= (trial ^ INT_MIN)).astype(jnp.float32),
                          axis=1, keepdims=True)
            prefix = jnp.where(cnt >= float(FG), trial, prefix)
        s_top = prefix ^ INT_MIN

        prefixb = jnp.zeros((B, 1), jnp.int32)
        for b in range(31, -1, -1):
            trial = prefixb | _low_mask(b)
            cnt = jnp.sum((key <= (trial ^ INT_MIN)).astype(jnp.float32),
                          axis=1, keepdims=True)
            prefixb = jnp.where(cnt >= float(BG), prefixb, prefixb | _bit(b))
        s_bot = prefixb ^ INT_MIN

        gt = key > s_top
        n_gt = jnp.sum(gt.astype(jnp.float32), axis=1, keepdims=True)
        sum_gt = jnp.sum(jnp.where(gt, sim, 0.0), axis=1, keepdims=True)
        v_top = lax.bitcast_convert_type(
            jnp.where(s_top >= 0, s_top, s_top ^ POS_MASK), jnp.float32)
        pos = (sum_gt + (FG - n_gt) * v_top) / FG

        lt = key < s_bot
        n_lt = jnp.sum(lt.astype(jnp.float32), axis=1, keepdims=True)
        sum_lt = jnp.sum(jnp.where(lt, sim, 0.0), axis=1, keepdims=True)
        v_bot = lax.bitcast_convert_type(
            jnp.where(s_bot >= 0, s_bot, s_bot ^ POS_MASK), jnp.float32)
        neg = (sum_lt + (BG - n_lt) * v_bot) / BG

        thr_ref[...] = jnp.broadcast_to(v_top, (B, 16))
        ngt_ref[...] = jnp.broadcast_to(n_gt.astype(jnp.int32), (B, 16))
        pos_ref[...] = jnp.broadcast_to(pos, (B, 16))
        neg_ref[...] = jnp.broadcast_to(neg, (B, 16))


def _run_stage_a(att_feat, aud_feat):
    aud3 = aud_feat.reshape(B, C, 1)
    return pl.pallas_call(
        _sim_topk_body,
        grid=(B // A1_ROWS,),
        in_specs=[
            pl.BlockSpec((A1_ROWS, C, K), lambda b: (b, 0, 0)),
            pl.BlockSpec((A1_ROWS, C, 1), lambda b: (b, 0, 0)),
        ],
        out_specs=[
            pl.BlockSpec((A1_ROWS, K), lambda b: (b, 0)),
            pl.BlockSpec((B, 16), lambda b: (0, 0)),
            pl.BlockSpec((B, 16), lambda b: (0, 0)),
            pl.BlockSpec((B, 16), lambda b: (0, 0)),
            pl.BlockSpec((B, 16), lambda b: (0, 0)),
        ],
        out_shape=[
            jax.ShapeDtypeStruct((B, K), jnp.float32),
            jax.ShapeDtypeStruct((B, 16), jnp.float32),
            jax.ShapeDtypeStruct((B, 16), jnp.int32),
            jax.ShapeDtypeStruct((B, 16), jnp.float32),
            jax.ShapeDtypeStruct((B, 16), jnp.float32),
        ],
        scratch_shapes=[pltpu.VMEM((B, K), jnp.float32)],
    )(att_feat, aud3)


def _sc_mask_body(sim_hbm, thr_hbm, ngt_hbm, out_hbm,
                  sim_v, thr_v, ngt_v, mask_v, sem):
    del sem
    cid = lax.axis_index("c")
    sid = lax.axis_index("s")
    wid = cid * NUM_SUBCORES + sid
    zeros16i = jnp.zeros((16,), jnp.int32)
    ones16f = jnp.full((16,), 1.0, jnp.float32)
    zeros16f = jnp.zeros((16,), jnp.float32)

    for i in range(B_PER_W):
        b = wid * B_PER_W + i
        pltpu.sync_copy(sim_hbm.at[b], sim_v)
        pltpu.sync_copy(thr_hbm.at[b], thr_v)
        pltpu.sync_copy(ngt_hbm.at[b], ngt_v)
        thr = thr_v[...]
        ngt = ngt_v[...]

        ceq = zeros16i
        for j in range(K // 16):
            svec = sim_v[pl.ds(j * 16, 16)]
            m_gt = svec > thr
            m_eq = svec == thr
            incl_e = plsc.cumsum(m_eq.astype(jnp.int32))
            tie_rank = ngt + ceq + incl_e - 1
            m_sel = jnp.logical_or(
                m_gt, jnp.logical_and(m_eq, tie_rank < FG))
            mask_v[pl.ds(j * 16, 16)] = jnp.where(m_sel, ones16f, zeros16f)
            ceq = ceq + plsc.all_reduce_population_count(m_eq)

        pltpu.sync_copy(mask_v, out_hbm.at[b])


def _run_stage_b1(sim, thr, ngt):
    mesh = plsc.VectorSubcoreMesh(core_axis_name="c", subcore_axis_name="s")
    fn = functools.partial(
        pl.kernel,
        out_type=jax.ShapeDtypeStruct((B, K), jnp.float32),
        mesh=mesh,
        compiler_params=pltpu.CompilerParams(needs_layout_passes=False),
        scratch_types=[
            pltpu.VMEM((K,), jnp.float32),
            pltpu.VMEM((16,), jnp.float32),
            pltpu.VMEM((16,), jnp.int32),
            pltpu.VMEM((K,), jnp.float32),
            pltpu.SemaphoreType.DMA,
        ],
    )(_sc_mask_body)
    return fn(sim, thr, ngt)


B2_ROWS = 4


def _hm_loss_body(hm_ref, mask_ref, pos_ref, neg_ref, av_ref,
                  dis_ref, div_ref, comb_ref):
    step = pl.program_id(0)
    for i in range(B2_ROWS):
        hm = hm_ref[i].reshape(HW, K)
        m = mask_ref[i]
        comb = lax.dot_general(m, hm, (((1,), (1,)), ((), ())),
                               preferred_element_type=jnp.float32)
        comb_ref[pl.ds(step * B2_ROWS + i, 1), :] = comb

    @pl.when(step == B // B2_ROWS - 1)
    def _():
        p = pos_ref[:, 0:1]
        n = neg_ref[:, 0:1]
        mx = jnp.maximum(p, n)
        lse = mx + jnp.log(jnp.exp(p - mx) + jnp.exp(n - mx))
        dis_ref[...] = jnp.mean(lse - p).reshape(1, 1)

        c = comb_ref[...] * (1.0 / FG)
        a = av_ref[...]
        cm = jnp.max(c, axis=1, keepdims=True)
        ce = jnp.exp(c - cm)
        cz = jnp.sum(ce, axis=1, keepdims=True)
        att = ce / cz
        log_att = (c - cm) - jnp.log(cz)
        am = jnp.max(a, axis=1, keepdims=True)
        ae = jnp.exp(a - am)
        az = jnp.sum(ae, axis=1, keepdims=True)
        avd = ae / az
        log_av = (a - am) - jnp.log(az)
        logm = jnp.log(0.5 * (att + avd))
        div = (jnp.sum(att * (log_att - logm)) +
               jnp.sum(avd * (log_av - logm))) / (2.0 * B)
        div_ref[...] = div.reshape(1, 1)


def _run_stage_b2c(hm_t, mask, pos, neg, av):
    return pl.pallas_call(
        _hm_loss_body,
        grid=(B // B2_ROWS,),
        in_specs=[
            pl.BlockSpec((B2_ROWS, H, W, K), lambda b: (b, 0, 0, 0)),
            pl.BlockSpec((B2_ROWS, 1, K), lambda b: (b, 0, 0)),
            pl.BlockSpec((B, 16), lambda b: (0, 0)),
            pl.BlockSpec((B, 16), lambda b: (0, 0)),
            pl.BlockSpec((B, HW), lambda b: (0, 0)),
        ],
        out_specs=[
            pl.BlockSpec((1, 1), lambda b: (0, 0)),
            pl.BlockSpec((1, 1), lambda b: (0, 0)),
        ],
        out_shape=[
            jax.ShapeDtypeStruct((1, 1), jnp.float32),
            jax.ShapeDtypeStruct((1, 1), jnp.float32),
        ],
        scratch_shapes=[pltpu.VMEM((B, HW), jnp.float32)],
    )(hm_t, mask.reshape(B, 1, K), pos, neg, av)


def kernel(att_feat, aud_feat, att_heatmaps, av_heatmaps):
    sim, thr, ngt, pos, neg = _run_stage_a(att_feat, aud_feat)
    mask = _run_stage_b1(sim, thr, ngt)
    hm_t = jnp.transpose(att_heatmaps, (0, 2, 3, 1))
    dis, div = _run_stage_b2c(hm_t, mask, pos, neg,
                              av_heatmaps.reshape(B, HW))
    return dis.reshape(()), div.reshape(())
